# baseline (speedup 1.0000x reference)
import functools

import jax
import jax.numpy as jnp
from jax.experimental import pallas as pl
from jax.experimental.pallas import tpu as pltpu

_BLOCK_ROWS = 1000


def _fused_linear_kernel(x_ref, wc_ref, bc_ref, wb_ref, bb_ref,
                         scores_ref, deltas_ref):
    x = x_ref[...]
    scores_ref[...] = (
        jnp.dot(x, wc_ref[...], preferred_element_type=jnp.float32)
        + bc_ref[...]
    )
    deltas_ref[...] = (
        jnp.dot(x, wb_ref[...], preferred_element_type=jnp.float32)
        + bb_ref[...]
    )


@jax.jit
def kernel(x, W_cls, b_cls, W_box, b_box):
    if x.ndim > 2:
        x = x.reshape(x.shape[0], -1)
    n, in_dim = x.shape
    n_cls = W_cls.shape[0]
    n_box = W_box.shape[0]

    wc_t = W_cls.T
    wb_t = W_box.T
    bc = b_cls.reshape(1, n_cls)
    bb = b_box.reshape(1, n_box)

    grid = (pl.cdiv(n, _BLOCK_ROWS),)
    scores, deltas = pl.pallas_call(
        _fused_linear_kernel,
        grid=grid,
        in_specs=[
            pl.BlockSpec((_BLOCK_ROWS, in_dim), lambda i: (i, 0)),
            pl.BlockSpec((in_dim, n_cls), lambda i: (0, 0)),
            pl.BlockSpec((1, n_cls), lambda i: (0, 0)),
            pl.BlockSpec((in_dim, n_box), lambda i: (0, 0)),
            pl.BlockSpec((1, n_box), lambda i: (0, 0)),
        ],
        out_specs=[
            pl.BlockSpec((_BLOCK_ROWS, n_cls), lambda i: (i, 0)),
            pl.BlockSpec((_BLOCK_ROWS, n_box), lambda i: (i, 0)),
        ],
        out_shape=[
            jax.ShapeDtypeStruct((n, n_cls), jnp.float32),
            jax.ShapeDtypeStruct((n, n_box), jnp.float32),
        ],
        compiler_params=pltpu.CompilerParams(
            dimension_semantics=("arbitrary",),
        ),
    )(x, wc_t, bc, wb_t, bb)
    return (scores, deltas)


# speedup vs baseline: 1.0025x; 1.0025x over previous
import functools

import jax
import jax.numpy as jnp
from jax.experimental import pallas as pl
from jax.experimental.pallas import tpu as pltpu

_BLOCK_ROWS = 1000


def _fused_linear_kernel(x_ref, wc_ref, bc_ref, wb_ref, bb_ref,
                         scores_ref, deltas_ref):
    x = x_ref[...]
    scores_ref[...] = (
        jnp.dot(x, wc_ref[...], preferred_element_type=jnp.float32)
        + bc_ref[...]
    )
    deltas_ref[...] = (
        jnp.dot(x, wb_ref[...], preferred_element_type=jnp.float32)
        + bb_ref[...]
    )


@jax.jit
def kernel(x, W_cls, b_cls, W_box, b_box):
    if x.ndim > 2:
        x = x.reshape(x.shape[0], -1)
    n, in_dim = x.shape
    n_cls = W_cls.shape[0]
    n_box = W_box.shape[0]

    wc_t = W_cls.T
    wb_t = W_box.T
    bc = b_cls.reshape(1, n_cls)
    bb = b_box.reshape(1, n_box)

    grid = (pl.cdiv(n, _BLOCK_ROWS),)
    scores, deltas = pl.pallas_call(
        _fused_linear_kernel,
        grid=grid,
        in_specs=[
            pl.BlockSpec((_BLOCK_ROWS, in_dim), lambda i: (i, 0)),
            pl.BlockSpec((in_dim, n_cls), lambda i: (0, 0)),
            pl.BlockSpec((1, n_cls), lambda i: (0, 0)),
            pl.BlockSpec((in_dim, n_box), lambda i: (0, 0)),
            pl.BlockSpec((1, n_box), lambda i: (0, 0)),
        ],
        out_specs=[
            pl.BlockSpec((_BLOCK_ROWS, n_cls), lambda i: (i, 0)),
            pl.BlockSpec((_BLOCK_ROWS, n_box), lambda i: (i, 0)),
        ],
        out_shape=[
            jax.ShapeDtypeStruct((n, n_cls), jnp.float32),
            jax.ShapeDtypeStruct((n, n_box), jnp.float32),
        ],
        compiler_params=pltpu.CompilerParams(
            dimension_semantics=("parallel",),
        ),
    )(x, wc_t, bc, wb_t, bb)
    return (scores, deltas)


# speedup vs baseline: 1.0536x; 1.0510x over previous
import functools

import jax
import jax.numpy as jnp
from jax.experimental import pallas as pl
from jax.experimental.pallas import tpu as pltpu

_BLOCK_ROWS = 2000


def _fused_linear_kernel(x_ref, wc_ref, bc_ref, wb_ref, bb_ref,
                         scores_ref, deltas_ref):
    x = x_ref[...]
    scores_ref[...] = (
        jnp.dot(x, wc_ref[...], preferred_element_type=jnp.float32)
        + bc_ref[...]
    )
    deltas_ref[...] = (
        jnp.dot(x, wb_ref[...], preferred_element_type=jnp.float32)
        + bb_ref[...]
    )


@jax.jit
def kernel(x, W_cls, b_cls, W_box, b_box):
    if x.ndim > 2:
        x = x.reshape(x.shape[0], -1)
    n, in_dim = x.shape
    n_cls = W_cls.shape[0]
    n_box = W_box.shape[0]

    wc_t = W_cls.T
    wb_t = W_box.T
    bc = b_cls.reshape(1, n_cls)
    bb = b_box.reshape(1, n_box)

    grid = (pl.cdiv(n, _BLOCK_ROWS),)
    scores, deltas = pl.pallas_call(
        _fused_linear_kernel,
        grid=grid,
        in_specs=[
            pl.BlockSpec((_BLOCK_ROWS, in_dim), lambda i: (i, 0)),
            pl.BlockSpec((in_dim, n_cls), lambda i: (0, 0)),
            pl.BlockSpec((1, n_cls), lambda i: (0, 0)),
            pl.BlockSpec((in_dim, n_box), lambda i: (0, 0)),
            pl.BlockSpec((1, n_box), lambda i: (0, 0)),
        ],
        out_specs=[
            pl.BlockSpec((_BLOCK_ROWS, n_cls), lambda i: (i, 0)),
            pl.BlockSpec((_BLOCK_ROWS, n_box), lambda i: (i, 0)),
        ],
        out_shape=[
            jax.ShapeDtypeStruct((n, n_cls), jnp.float32),
            jax.ShapeDtypeStruct((n, n_box), jnp.float32),
        ],
        compiler_params=pltpu.CompilerParams(
            dimension_semantics=("parallel",),
        ),
    )(x, wc_t, bc, wb_t, bb)
    return (scores, deltas)
